# baseline (speedup 1.0000x reference)
import dataclasses
import functools

import jax
import jax.numpy as jnp
from jax import lax
from jax.experimental import pallas as pl
from jax.experimental.pallas import tpu as pltpu
from jax.experimental.pallas import tpu_sc as plsc

_NC = 2
_NS = 16
_LANES = 16
_NW = _NC * _NS
_K = 128


def _cdiv(a, b):
    return (a + b - 1) // b


def _sc_compiler_params():
    cp = pltpu.CompilerParams()
    if "needs_layout_passes" in pltpu.CompilerParams.__dataclass_fields__:
        cp = dataclasses.replace(cp, needs_layout_passes=False)
    return cp


_ROW_BLK = 1000


def _row_spec(d):
    return pl.BlockSpec((_ROW_BLK, d), lambda i: (i, 0))


def _full_spec(r, c):
    return pl.BlockSpec((r, c), lambda i: (0, 0))


def _proj(xp, att):
    return lax.dot_general(
        xp, att, (((1,), (1,)), ((), ())),
        preferred_element_type=jnp.float32,
    )


def _agg(a0_ref, a1_ref, dn_ref):
    dn = (dn_ref[:, 0] + dn_ref[:, 1])[:, None]
    acc = a0_ref[...] + a1_ref[...]
    safe = jnp.where(dn > 0, dn, 1.0)
    return jnp.where(dn > 0, acc / safe, 0.0)


def _inpre_body(x_ref, w1_ref, b1_ref, wg_ref, att_ref, h_ref, xp_ref, a_ref):
    h = jnp.maximum(
        jnp.dot(x_ref[...], w1_ref[...], preferred_element_type=jnp.float32)
        + b1_ref[...], 0.0)
    h_ref[...] = h
    xp = jnp.dot(h, wg_ref[...], preferred_element_type=jnp.float32)
    xp_ref[...] = xp
    a_ref[...] = _proj(xp, att_ref[...])


def _tc_input_pre(x, W1, b1, Wg, att2):
    n, d = x.shape
    return pl.pallas_call(
        _inpre_body,
        grid=(n // _ROW_BLK,),
        in_specs=[_row_spec(d), _full_spec(d, d), _full_spec(1, d),
                  _full_spec(d, d), _full_spec(2, d)],
        out_specs=[_row_spec(d), _row_spec(d), _row_spec(2)],
        out_shape=[
            jax.ShapeDtypeStruct((n, d), jnp.float32),
            jax.ShapeDtypeStruct((n, d), jnp.float32),
            jax.ShapeDtypeStruct((n, 2), jnp.float32),
        ],
    )(x, W1, b1, Wg, att2)


def _epipre_body(h_ref, a0_ref, a1_ref, dn_ref, bg_ref, wg_ref, att_ref,
                 hn_ref, xp_ref, a_ref):
    h = jnp.maximum(h_ref[...] + _agg(a0_ref, a1_ref, dn_ref) + bg_ref[...],
                    0.0)
    hn_ref[...] = h
    xp = jnp.dot(h, wg_ref[...], preferred_element_type=jnp.float32)
    xp_ref[...] = xp
    a_ref[...] = _proj(xp, att_ref[...])


def _tc_epi_pre(h, acc0, acc1, dnT, bg, Wg, att2):
    n, d = h.shape
    return pl.pallas_call(
        _epipre_body,
        grid=(n // _ROW_BLK,),
        in_specs=[_row_spec(d), _row_spec(d), _row_spec(d), _row_spec(2),
                  _full_spec(1, d), _full_spec(d, d), _full_spec(2, d)],
        out_specs=[_row_spec(d), _row_spec(d), _row_spec(2)],
        out_shape=[
            jax.ShapeDtypeStruct((n, d), jnp.float32),
            jax.ShapeDtypeStruct((n, d), jnp.float32),
            jax.ShapeDtypeStruct((n, 2), jnp.float32),
        ],
    )(h, acc0, acc1, dnT, bg, Wg, att2)


def _epifin_body(h_ref, a0_ref, a1_ref, dn_ref, bg_ref,
                 wp1_ref, bp1_ref, wp2_ref, bp2_ref, wea_ref, web_ref,
                 be1_ref, inv_n_ref, np_ref, p_ref, q_ref, g_ref):
    h = jnp.maximum(h_ref[...] + _agg(a0_ref, a1_ref, dn_ref) + bg_ref[...],
                    0.0)
    t = jnp.maximum(
        jnp.dot(h, wp1_ref[...], preferred_element_type=jnp.float32)
        + bp1_ref[...], 0.0)
    np_ref[...] = (
        jnp.dot(t, wp2_ref[...], preferred_element_type=jnp.float32)
        + bp2_ref[...])
    p_ref[...] = (
        jnp.dot(h, wea_ref[...], preferred_element_type=jnp.float32)
        + be1_ref[...])
    q_ref[...] = jnp.dot(h, web_ref[...], preferred_element_type=jnp.float32)
    i = pl.program_id(0)

    @pl.when(i == 0)
    def _():
        g_ref[...] = jnp.zeros_like(g_ref)

    g_ref[...] += jnp.sum(h, axis=0, keepdims=True) * inv_n_ref[...]


def _tc_epi_final(h, acc0, acc1, dnT, bg, Wp1, bp1, Wp2, bp2, We1a, We1b, be1):
    n, d = h.shape
    inv_n = jnp.full((1, 1), 1.0 / n, jnp.float32)
    return pl.pallas_call(
        _epifin_body,
        grid=(n // _ROW_BLK,),
        in_specs=[_row_spec(d), _row_spec(d), _row_spec(d), _row_spec(2),
                  _full_spec(1, d),
                  _full_spec(d, d), _full_spec(1, d), _full_spec(d, d),
                  _full_spec(1, d), _full_spec(d, d), _full_spec(d, d),
                  _full_spec(1, d), _full_spec(1, 1)],
        out_specs=[_row_spec(d), _row_spec(d), _row_spec(d),
                   _full_spec(1, d)],
        out_shape=[
            jax.ShapeDtypeStruct((n, d), jnp.float32),
            jax.ShapeDtypeStruct((n, d), jnp.float32),
            jax.ShapeDtypeStruct((n, d), jnp.float32),
            jax.ShapeDtypeStruct((1, d), jnp.float32),
        ],
    )(h, acc0, acc1, dnT, bg, Wp1, bp1, Wp2, bp2, We1a, We1b, be1, inv_n)


_KG = 64
_DEPTH = 4


@functools.lru_cache(maxsize=None)
def _sc_gat_kernel(n, e, d):
    rpt = _K * _cdiv(_cdiv(n, _NS), _K)
    npad = _NS * rpt
    nblk = e // _KG
    t_steps = _cdiv(nblk, _NW)
    nch = d // _LANES

    mesh = plsc.VectorSubcoreMesh(core_axis_name="c", subcore_axis_name="s")

    def body(xp_hbm, as_hbm, ad_hbm, g_hbm, eidx_hbm, acc_out, den_out,
             gvv, *rest):
        bufs = rest[:5 * _DEPTH]
        acc_sh, den_sh = rest[5 * _DEPTH:5 * _DEPTH + 2]
        sems = rest[5 * _DEPTH + 2:]
        slots = tuple(
            tuple(bufs[q + _DEPTH * k] for k in range(5))
            + (sems[q], sems[_DEPTH + q])
            for q in range(_DEPTH))

        cid = lax.axis_index("c")
        sid = lax.axis_index("s")
        wid = cid * _NS + sid

        pltpu.sync_copy(g_hbm, gvv)

        z16 = jnp.zeros((_LANES,), jnp.float32)
        rows0 = slots[0][4]
        exb0 = slots[0][3]

        @pl.loop(0, _KG)
        def _(r):
            for c in range(nch):
                rows0[r, pl.ds(c * _LANES, _LANES)] = z16

        for i in range(_KG // _LANES):
            exb0[pl.ds(i * _LANES, _LANES)] = z16

        zbase = sid * rpt
        for k in range(rpt // _KG):
            pltpu.sync_copy(rows0, acc_sh.at[pl.ds(zbase + k * _KG, _KG)])
            pltpu.sync_copy(exb0, den_sh.at[pl.ds(zbase + k * _KG, _KG)])
        plsc.subcore_barrier()

        gvec = gvv[...]
        nb = (nblk - 1 - wid) // _NW + 1

        def issue(slot, t):
            sdb, avs, avd, exb, rows, gsem, ssem = slots[slot]
            @pl.when(t >= _DEPTH)
            def _():
                pltpu.make_async_copy(rows, acc_sh.at[sdb.at[1]], ssem).wait()
                pltpu.make_async_copy(exb, den_sh.at[sdb.at[1]], ssem).wait()
            blk = t * _NW + wid
            pltpu.sync_copy(eidx_hbm.at[blk], sdb)
            pltpu.async_copy(as_hbm.at[sdb.at[0]], avs, gsem)
            pltpu.async_copy(ad_hbm.at[sdb.at[1]], avd, gsem)
            pltpu.async_copy(xp_hbm.at[sdb.at[0]], rows, gsem)

        def compute(slot):
            sdb, avs, avd, exb, rows, gsem, ssem = slots[slot]
            pltpu.make_async_copy(as_hbm.at[sdb.at[0]], avs, gsem).wait()
            pltpu.make_async_copy(ad_hbm.at[sdb.at[1]], avd, gsem).wait()
            for j in range(_KG // _LANES):
                sl = pl.ds(j * _LANES, _LANES)
                ev = avs[sl] + avd[sl]
                ev = jnp.where(ev >= 0.0, ev, ev * 0.2)
                exb[sl] = jnp.exp(ev - gvec)
            pltpu.async_copy(exb, den_sh.at[sdb.at[1]], ssem, add=True)
            pltpu.make_async_copy(xp_hbm.at[sdb.at[0]], rows, gsem).wait()

            @pl.loop(0, _KG // _LANES)
            def _(jg):
                for rr in range(_LANES):
                    r = jg * _LANES + rr
                    exr = plsc.load_gather(
                        exb, [jnp.zeros((_LANES,), jnp.int32) + r])
                    for c in range(nch):
                        sl = pl.ds(c * _LANES, _LANES)
                        rows[r, sl] = rows[r, sl] * exr

            pltpu.async_copy(rows, acc_sh.at[sdb.at[1]], ssem, add=True)

        @pl.loop(0, (t_steps + 2 + _DEPTH - 1) // _DEPTH)
        def _(i):
            for q in range(_DEPTH):
                t = i * _DEPTH + q

                @pl.when(t < nb)
                def _(t=t, q=q):
                    issue(q, t)

                @pl.when((t >= 2) & (t - 2 < nb))
                def _(t=t, q=q):
                    compute((q + 2) % _DEPTH)

        for q in range(_DEPTH):
            sdb, avs, avd, exb, rows, gsem, ssem = slots[q]
            pltpu.make_async_copy(rows, acc_sh.at[sdb.at[1]], ssem).wait()
            pltpu.make_async_copy(exb, den_sh.at[sdb.at[1]], ssem).wait()

        plsc.subcore_barrier()
        ob = cid * npad + sid * rpt
        for k in range(rpt // _K):
            pltpu.sync_copy(acc_sh.at[pl.ds(zbase + k * _K, _K)],
                            acc_out.at[pl.ds(ob + k * _K, _K)])
        pltpu.sync_copy(den_sh.at[pl.ds(zbase, rpt)], den_out.at[pl.ds(ob, rpt)])

    kern = pl.kernel(
        body,
        out_type=[
            jax.ShapeDtypeStruct((_NC * npad, d), jnp.float32),
            jax.ShapeDtypeStruct((_NC * npad,), jnp.float32),
        ],
        mesh=mesh,
        scratch_types=(
            [pltpu.VMEM((_LANES,), jnp.float32)]
            + [pltpu.VMEM((2, _KG), jnp.int32)] * _DEPTH
            + [pltpu.VMEM((_KG,), jnp.float32)] * _DEPTH
            + [pltpu.VMEM((_KG,), jnp.float32)] * _DEPTH
            + [pltpu.VMEM((_KG,), jnp.float32)] * _DEPTH
            + [pltpu.VMEM((_KG, d), jnp.float32)] * _DEPTH
            + [pltpu.VMEM_SHARED((npad, d), jnp.float32),
               pltpu.VMEM_SHARED((npad,), jnp.float32)]
            + [pltpu.SemaphoreType.DMA] * (2 * _DEPTH)
        ),
        compiler_params=_sc_compiler_params(),
    )
    return kern, npad


@functools.lru_cache(maxsize=None)
def _sc_edge_kernel(n, e, d):
    nblk = e // _K
    t_steps = _cdiv(nblk, _NW)
    nch = d // _LANES

    mesh = plsc.VectorSubcoreMesh(core_axis_name="c", subcore_axis_name="s")

    def body(p_hbm, q_hbm, w2_hbm, be2_hbm, eidx_hbm, ep_out,
             sdb0, sdb1, prow0, prow1, qrow0, qrow1, outb0, outb1, w2v, be2v,
             gsem0, gsem1, osem0, osem1):
        cid = lax.axis_index("c")
        sid = lax.axis_index("s")
        wid = cid * _NS + sid

        pltpu.sync_copy(w2_hbm, w2v)
        pltpu.sync_copy(be2_hbm, be2v)
        be2r = be2v[...]
        lane = lax.iota(jnp.int32, _LANES)
        w2c = [[w2v[j, pl.ds(c * _LANES, _LANES)] for c in range(nch)]
               for j in range(3)]

        slots = ((sdb0, prow0, qrow0, outb0, gsem0, osem0),
                 (sdb1, prow1, qrow1, outb1, gsem1, osem1))
        nb = (nblk - 1 - wid) // _NW + 1

        def start(slot, t):
            sdb, prow, qrow, outb, gsem, osem = slots[slot]
            blk = t * _NW + wid
            pltpu.sync_copy(eidx_hbm.at[blk], sdb)
            pltpu.async_copy(p_hbm.at[sdb.at[0]], prow, gsem)
            pltpu.async_copy(q_hbm.at[sdb.at[1]], qrow, gsem)

        def finish(slot, t, drain):
            sdb, prow, qrow, outb, gsem, osem = slots[slot]
            pltpu.make_async_copy(p_hbm.at[sdb.at[0]], prow, gsem).wait()
            pltpu.make_async_copy(q_hbm.at[sdb.at[1]], qrow, gsem).wait()
            off = (t * _NW + wid) * _K
            if drain is True:
                pltpu.make_async_copy(outb, ep_out.at[pl.ds(off, _K)],
                                      osem).wait()
            elif drain is not False:
                @pl.when(drain)
                def _():
                    pltpu.make_async_copy(outb, ep_out.at[pl.ds(off, _K)],
                                          osem).wait()

            @pl.loop(0, _K // _LANES)
            def _(jg):
                for rr in range(_LANES):
                    r = jg * _LANES + rr
                    a0 = jnp.zeros((_LANES,), jnp.float32)
                    a1 = a0
                    a2 = a0
                    for c in range(nch):
                        sl = pl.ds(c * _LANES, _LANES)
                        tv = jnp.maximum(prow[r, sl] + qrow[r, sl], 0.0)
                        a0 = a0 + tv * w2c[0][c]
                        a1 = a1 + tv * w2c[1][c]
                        a2 = a2 + tv * w2c[2][c]
                    d0 = jnp.sum(a0)
                    d1 = jnp.sum(a1)
                    d2 = jnp.sum(a2)
                    orow = (be2r
                            + jnp.where(lane == 0, d0, 0.0)
                            + jnp.where(lane == 1, d1, 0.0)
                            + jnp.where(lane == 2, d2, 0.0))
                    outb[r, :] = orow

            pltpu.async_copy(outb, ep_out.at[pl.ds(off, _K)], osem)

        start(0, 0)

        @pl.loop(0, (t_steps + 1) // 2)
        def _(i):
            t0 = i * 2
            t1 = t0 + 1

            @pl.when(t1 < nb)
            def _():
                start(1, t1)

            @pl.when(t0 < nb)
            def _():
                finish(0, t0, drain=t0 >= 2)

            @pl.when(t0 + 2 < nb)
            def _():
                start(0, t0 + 2)

            @pl.when(t1 < nb)
            def _():
                finish(1, t1, drain=t1 >= 3)

        def odrain(slot):
            sdb, prow, qrow, outb, gsem, osem = slots[slot]
            pltpu.make_async_copy(outb, ep_out.at[pl.ds(0, _K)], osem).wait()

        odrain(0)

        @pl.when(nb >= 2)
        def _():
            odrain(1)

    kern = pl.kernel(
        body,
        out_type=jax.ShapeDtypeStruct((e, _LANES), jnp.float32),
        mesh=mesh,
        scratch_types=[
            pltpu.VMEM((2, _K), jnp.int32),
            pltpu.VMEM((2, _K), jnp.int32),
            pltpu.VMEM((_K, d), jnp.float32),
            pltpu.VMEM((_K, d), jnp.float32),
            pltpu.VMEM((_K, d), jnp.float32),
            pltpu.VMEM((_K, d), jnp.float32),
            pltpu.VMEM((_K, _LANES), jnp.float32),
            pltpu.VMEM((_K, _LANES), jnp.float32),
            pltpu.VMEM((3, d), jnp.float32),
            pltpu.VMEM((_LANES,), jnp.float32),
            pltpu.SemaphoreType.DMA,
            pltpu.SemaphoreType.DMA,
            pltpu.SemaphoreType.DMA,
            pltpu.SemaphoreType.DMA,
        ],
        compiler_params=_sc_compiler_params(),
    )
    return kern


def kernel(x, edge_index, W1, b1, gat_Wg, gat_att_src, gat_att_dst, gat_bias,
           Wp1, bp1, Wp2, bp2, We1, be1, We2, be2):
    n, d = x.shape
    e = edge_index.shape[1]
    num_layers = gat_Wg.shape[0]

    eidx32 = edge_index.astype(jnp.int32)
    eidx3 = eidx32.reshape(2, e // _K, _K).transpose(1, 0, 2)
    eidxg = eidx32.reshape(2, e // _KG, _KG).transpose(1, 0, 2)

    gat_kern, npad = _sc_gat_kernel(n, e, d)

    def sc_layer(l, xp, aT):
        a_s = aT[:, 0]
        a_d = aT[:, 1]
        gmax = jnp.max(a_s) + jnp.max(a_d)
        gmax = jnp.where(gmax >= 0.0, gmax, 0.2 * gmax)
        garr = jnp.full((_LANES,), gmax, jnp.float32)
        accs, dens = gat_kern(xp, a_s, a_d, garr, eidxg)
        acc0 = accs[:n]
        acc1 = accs[npad:npad + n]
        dnT = dens.reshape(_NC, npad)[:, :n].T
        return acc0, acc1, dnT

    def att2(l):
        return jnp.stack([gat_att_src[l], gat_att_dst[l]])

    h, xp, aT = _tc_input_pre(x, W1, b1.reshape(1, d), gat_Wg[0], att2(0))
    for l in range(num_layers - 1):
        acc0, acc1, dnT = sc_layer(l, xp, aT)
        h, xp, aT = _tc_epi_pre(h, acc0, acc1, dnT, gat_bias[l].reshape(1, d),
                                gat_Wg[l + 1], att2(l + 1))
    acc0, acc1, dnT = sc_layer(num_layers - 1, xp, aT)

    node_pred, P, Q, graph_emb = _tc_epi_final(
        h, acc0, acc1, dnT, gat_bias[num_layers - 1].reshape(1, d),
        Wp1, bp1.reshape(1, d), Wp2, bp2.reshape(1, d),
        We1[:d], We1[d:], be1.reshape(1, d))

    edge_kern = _sc_edge_kernel(n, e, d)
    w2t = We2.T
    be2p = jnp.concatenate([be2, jnp.zeros((_LANES - 3,), jnp.float32)])
    ep16 = edge_kern(P, Q, w2t, be2p, eidx3)
    edge_pred = ep16[:, :3]

    return (node_pred, edge_pred, graph_emb)


# speedup vs baseline: 1.0036x; 1.0036x over previous
import dataclasses
import functools

import jax
import jax.numpy as jnp
from jax import lax
from jax.experimental import pallas as pl
from jax.experimental.pallas import tpu as pltpu
from jax.experimental.pallas import tpu_sc as plsc

_NC = 2
_NS = 16
_LANES = 16
_NW = _NC * _NS
_K = 128


def _cdiv(a, b):
    return (a + b - 1) // b


def _sc_compiler_params():
    cp = pltpu.CompilerParams()
    if "needs_layout_passes" in pltpu.CompilerParams.__dataclass_fields__:
        cp = dataclasses.replace(cp, needs_layout_passes=False)
    return cp


_ROW_BLK = 1000


def _row_spec(d):
    return pl.BlockSpec((_ROW_BLK, d), lambda i: (i, 0))


def _full_spec(r, c):
    return pl.BlockSpec((r, c), lambda i: (0, 0))


def _proj(xp, att):
    return lax.dot_general(
        xp, att, (((1,), (1,)), ((), ())),
        preferred_element_type=jnp.float32,
    )


def _agg(a0_ref, a1_ref, dn_ref):
    dn = (dn_ref[:, 0] + dn_ref[:, 1])[:, None]
    acc = a0_ref[...] + a1_ref[...]
    safe = jnp.where(dn > 0, dn, 1.0)
    return jnp.where(dn > 0, acc / safe, 0.0)


def _inpre_body(x_ref, w1_ref, b1_ref, wg_ref, att_ref, h_ref, xp_ref, a_ref):
    h = jnp.maximum(
        jnp.dot(x_ref[...], w1_ref[...], preferred_element_type=jnp.float32)
        + b1_ref[...], 0.0)
    h_ref[...] = h
    xp = jnp.dot(h, wg_ref[...], preferred_element_type=jnp.float32)
    xp_ref[...] = xp
    a_ref[...] = _proj(xp, att_ref[...])


def _tc_input_pre(x, W1, b1, Wg, att2):
    n, d = x.shape
    return pl.pallas_call(
        _inpre_body,
        grid=(n // _ROW_BLK,),
        in_specs=[_row_spec(d), _full_spec(d, d), _full_spec(1, d),
                  _full_spec(d, d), _full_spec(2, d)],
        out_specs=[_row_spec(d), _row_spec(d), _row_spec(2)],
        out_shape=[
            jax.ShapeDtypeStruct((n, d), jnp.float32),
            jax.ShapeDtypeStruct((n, d), jnp.float32),
            jax.ShapeDtypeStruct((n, 2), jnp.float32),
        ],
    )(x, W1, b1, Wg, att2)


def _epipre_body(h_ref, a0_ref, a1_ref, dn_ref, bg_ref, wg_ref, att_ref,
                 hn_ref, xp_ref, a_ref):
    h = jnp.maximum(h_ref[...] + _agg(a0_ref, a1_ref, dn_ref) + bg_ref[...],
                    0.0)
    hn_ref[...] = h
    xp = jnp.dot(h, wg_ref[...], preferred_element_type=jnp.float32)
    xp_ref[...] = xp
    a_ref[...] = _proj(xp, att_ref[...])


def _tc_epi_pre(h, acc0, acc1, dnT, bg, Wg, att2):
    n, d = h.shape
    return pl.pallas_call(
        _epipre_body,
        grid=(n // _ROW_BLK,),
        in_specs=[_row_spec(d), _row_spec(d), _row_spec(d), _row_spec(2),
                  _full_spec(1, d), _full_spec(d, d), _full_spec(2, d)],
        out_specs=[_row_spec(d), _row_spec(d), _row_spec(2)],
        out_shape=[
            jax.ShapeDtypeStruct((n, d), jnp.float32),
            jax.ShapeDtypeStruct((n, d), jnp.float32),
            jax.ShapeDtypeStruct((n, 2), jnp.float32),
        ],
    )(h, acc0, acc1, dnT, bg, Wg, att2)


def _epipq_body(h_ref, a0_ref, a1_ref, dn_ref, bg_ref, wea_ref, web_ref,
                be1_ref, h3_ref, p_ref, q_ref):
    h = jnp.maximum(h_ref[...] + _agg(a0_ref, a1_ref, dn_ref) + bg_ref[...],
                    0.0)
    h3_ref[...] = h
    p_ref[...] = (
        jnp.dot(h, wea_ref[...], preferred_element_type=jnp.float32)
        + be1_ref[...])
    q_ref[...] = jnp.dot(h, web_ref[...], preferred_element_type=jnp.float32)


def _tc_epi_pq(h, acc0, acc1, dnT, bg, We1a, We1b, be1):
    n, d = h.shape
    return pl.pallas_call(
        _epipq_body,
        grid=(n // _ROW_BLK,),
        in_specs=[_row_spec(d), _row_spec(d), _row_spec(d), _row_spec(2),
                  _full_spec(1, d), _full_spec(d, d), _full_spec(d, d),
                  _full_spec(1, d)],
        out_specs=[_row_spec(d), _row_spec(d), _row_spec(d)],
        out_shape=[
            jax.ShapeDtypeStruct((n, d), jnp.float32),
            jax.ShapeDtypeStruct((n, d), jnp.float32),
            jax.ShapeDtypeStruct((n, d), jnp.float32),
        ],
    )(h, acc0, acc1, dnT, bg, We1a, We1b, be1)


def _nodefin_body(h_ref, wp1_ref, bp1_ref, wp2_ref, bp2_ref, inv_n_ref,
                  np_ref, g_ref):
    h = h_ref[...]
    t = jnp.maximum(
        jnp.dot(h, wp1_ref[...], preferred_element_type=jnp.float32)
        + bp1_ref[...], 0.0)
    np_ref[...] = (
        jnp.dot(t, wp2_ref[...], preferred_element_type=jnp.float32)
        + bp2_ref[...])
    i = pl.program_id(0)

    @pl.when(i == 0)
    def _():
        g_ref[...] = jnp.zeros_like(g_ref)

    g_ref[...] += jnp.sum(h, axis=0, keepdims=True) * inv_n_ref[...]


def _tc_node_final(h3, Wp1, bp1, Wp2, bp2):
    n, d = h3.shape
    inv_n = jnp.full((1, 1), 1.0 / n, jnp.float32)
    return pl.pallas_call(
        _nodefin_body,
        grid=(n // _ROW_BLK,),
        in_specs=[_row_spec(d), _full_spec(d, d), _full_spec(1, d),
                  _full_spec(d, d), _full_spec(1, d), _full_spec(1, 1)],
        out_specs=[_row_spec(d), _full_spec(1, d)],
        out_shape=[
            jax.ShapeDtypeStruct((n, d), jnp.float32),
            jax.ShapeDtypeStruct((1, d), jnp.float32),
        ],
    )(h3, Wp1, bp1, Wp2, bp2, inv_n)


_KG = 64
_DEPTH = 4


@functools.lru_cache(maxsize=None)
def _sc_gat_kernel(n, e, d):
    rpt = _K * _cdiv(_cdiv(n, _NS), _K)
    npad = _NS * rpt
    nblk = e // _KG
    t_steps = _cdiv(nblk, _NW)
    nch = d // _LANES

    mesh = plsc.VectorSubcoreMesh(core_axis_name="c", subcore_axis_name="s")

    def body(xp_hbm, as_hbm, ad_hbm, g_hbm, eidx_hbm, acc_out, den_out,
             gvv, *rest):
        bufs = rest[:5 * _DEPTH]
        acc_sh, den_sh = rest[5 * _DEPTH:5 * _DEPTH + 2]
        sems = rest[5 * _DEPTH + 2:]
        slots = tuple(
            tuple(bufs[q + _DEPTH * k] for k in range(5))
            + (sems[q], sems[_DEPTH + q])
            for q in range(_DEPTH))

        cid = lax.axis_index("c")
        sid = lax.axis_index("s")
        wid = cid * _NS + sid

        pltpu.sync_copy(g_hbm, gvv)

        z16 = jnp.zeros((_LANES,), jnp.float32)
        rows0 = slots[0][4]
        exb0 = slots[0][3]

        @pl.loop(0, _KG)
        def _(r):
            for c in range(nch):
                rows0[r, pl.ds(c * _LANES, _LANES)] = z16

        for i in range(_KG // _LANES):
            exb0[pl.ds(i * _LANES, _LANES)] = z16

        zbase = sid * rpt
        zcp = []
        for k in range(rpt // _KG):
            zcp.append(pltpu.async_copy(
                rows0, acc_sh.at[pl.ds(zbase + k * _KG, _KG)], sems[0]))
            zcp.append(pltpu.async_copy(
                exb0, den_sh.at[pl.ds(zbase + k * _KG, _KG)], sems[1]))
        for c in zcp:
            c.wait()
        plsc.subcore_barrier()

        gvec = gvv[...]
        nb = (nblk - 1 - wid) // _NW + 1

        def issue(slot, t):
            sdb, avs, avd, exb, rows, gsem, ssem = slots[slot]
            @pl.when(t >= _DEPTH)
            def _():
                pltpu.make_async_copy(rows, acc_sh.at[sdb.at[1]], ssem).wait()
                pltpu.make_async_copy(exb, den_sh.at[sdb.at[1]], ssem).wait()
            blk = t * _NW + wid
            pltpu.sync_copy(eidx_hbm.at[blk], sdb)
            pltpu.async_copy(as_hbm.at[sdb.at[0]], avs, gsem)
            pltpu.async_copy(ad_hbm.at[sdb.at[1]], avd, gsem)
            pltpu.async_copy(xp_hbm.at[sdb.at[0]], rows, gsem)

        def compute(slot):
            sdb, avs, avd, exb, rows, gsem, ssem = slots[slot]
            pltpu.make_async_copy(as_hbm.at[sdb.at[0]], avs, gsem).wait()
            pltpu.make_async_copy(ad_hbm.at[sdb.at[1]], avd, gsem).wait()
            for j in range(_KG // _LANES):
                sl = pl.ds(j * _LANES, _LANES)
                ev = avs[sl] + avd[sl]
                ev = jnp.where(ev >= 0.0, ev, ev * 0.2)
                exb[sl] = jnp.exp(ev - gvec)
            pltpu.async_copy(exb, den_sh.at[sdb.at[1]], ssem, add=True)
            pltpu.make_async_copy(xp_hbm.at[sdb.at[0]], rows, gsem).wait()

            @pl.loop(0, _KG // _LANES)
            def _(jg):
                for rr in range(_LANES):
                    r = jg * _LANES + rr
                    exr = plsc.load_gather(
                        exb, [jnp.zeros((_LANES,), jnp.int32) + r])
                    for c in range(nch):
                        sl = pl.ds(c * _LANES, _LANES)
                        rows[r, sl] = rows[r, sl] * exr

            pltpu.async_copy(rows, acc_sh.at[sdb.at[1]], ssem, add=True)

        @pl.loop(0, (t_steps + 2 + _DEPTH - 1) // _DEPTH)
        def _(i):
            for q in range(_DEPTH):
                t = i * _DEPTH + q

                @pl.when(t < nb)
                def _(t=t, q=q):
                    issue(q, t)

                @pl.when((t >= 2) & (t - 2 < nb))
                def _(t=t, q=q):
                    compute((q + 2) % _DEPTH)

        for q in range(_DEPTH):
            sdb, avs, avd, exb, rows, gsem, ssem = slots[q]
            pltpu.make_async_copy(rows, acc_sh.at[sdb.at[1]], ssem).wait()
            pltpu.make_async_copy(exb, den_sh.at[sdb.at[1]], ssem).wait()

        plsc.subcore_barrier()
        ob = cid * npad + sid * rpt
        ocp = []
        for k in range(rpt // _K):
            ocp.append(pltpu.async_copy(
                acc_sh.at[pl.ds(zbase + k * _K, _K)],
                acc_out.at[pl.ds(ob + k * _K, _K)], sems[0]))
        ocp.append(pltpu.async_copy(
            den_sh.at[pl.ds(zbase, rpt)], den_out.at[pl.ds(ob, rpt)], sems[1]))
        for c in ocp:
            c.wait()

    kern = pl.kernel(
        body,
        out_type=[
            jax.ShapeDtypeStruct((_NC * npad, d), jnp.float32),
            jax.ShapeDtypeStruct((_NC * npad,), jnp.float32),
        ],
        mesh=mesh,
        scratch_types=(
            [pltpu.VMEM((_LANES,), jnp.float32)]
            + [pltpu.VMEM((2, _KG), jnp.int32)] * _DEPTH
            + [pltpu.VMEM((_KG,), jnp.float32)] * _DEPTH
            + [pltpu.VMEM((_KG,), jnp.float32)] * _DEPTH
            + [pltpu.VMEM((_KG,), jnp.float32)] * _DEPTH
            + [pltpu.VMEM((_KG, d), jnp.float32)] * _DEPTH
            + [pltpu.VMEM_SHARED((npad, d), jnp.float32),
               pltpu.VMEM_SHARED((npad,), jnp.float32)]
            + [pltpu.SemaphoreType.DMA] * (2 * _DEPTH)
        ),
        compiler_params=_sc_compiler_params(),
    )
    return kern, npad


@functools.lru_cache(maxsize=None)
def _sc_edge_kernel(n, e, d):
    nblk = e // _K
    t_steps = _cdiv(nblk, _NW)
    nch = d // _LANES

    mesh = plsc.VectorSubcoreMesh(core_axis_name="c", subcore_axis_name="s")

    def body(p_hbm, q_hbm, w2_hbm, be2_hbm, eidx_hbm, ep_out,
             sdb0, sdb1, prow0, prow1, qrow0, qrow1, outb0, outb1, w2v, be2v,
             gsem0, gsem1, osem0, osem1):
        cid = lax.axis_index("c")
        sid = lax.axis_index("s")
        wid = cid * _NS + sid

        pltpu.sync_copy(w2_hbm, w2v)
        pltpu.sync_copy(be2_hbm, be2v)
        be2r = be2v[...]
        lane = lax.iota(jnp.int32, _LANES)
        w2c = [[w2v[j, pl.ds(c * _LANES, _LANES)] for c in range(nch)]
               for j in range(3)]

        slots = ((sdb0, prow0, qrow0, outb0, gsem0, osem0),
                 (sdb1, prow1, qrow1, outb1, gsem1, osem1))
        nb = (nblk - 1 - wid) // _NW + 1

        def start(slot, t):
            sdb, prow, qrow, outb, gsem, osem = slots[slot]
            blk = t * _NW + wid
            pltpu.sync_copy(eidx_hbm.at[blk], sdb)
            pltpu.async_copy(p_hbm.at[sdb.at[0]], prow, gsem)
            pltpu.async_copy(q_hbm.at[sdb.at[1]], qrow, gsem)

        def finish(slot, t, drain):
            sdb, prow, qrow, outb, gsem, osem = slots[slot]
            pltpu.make_async_copy(p_hbm.at[sdb.at[0]], prow, gsem).wait()
            pltpu.make_async_copy(q_hbm.at[sdb.at[1]], qrow, gsem).wait()
            off = (t * _NW + wid) * _K
            if drain is True:
                pltpu.make_async_copy(outb, ep_out.at[pl.ds(off, _K)],
                                      osem).wait()
            elif drain is not False:
                @pl.when(drain)
                def _():
                    pltpu.make_async_copy(outb, ep_out.at[pl.ds(off, _K)],
                                          osem).wait()

            @pl.loop(0, _K // _LANES)
            def _(jg):
                for rr in range(_LANES):
                    r = jg * _LANES + rr
                    a0 = jnp.zeros((_LANES,), jnp.float32)
                    a1 = a0
                    a2 = a0
                    for c in range(nch):
                        sl = pl.ds(c * _LANES, _LANES)
                        tv = jnp.maximum(prow[r, sl] + qrow[r, sl], 0.0)
                        a0 = a0 + tv * w2c[0][c]
                        a1 = a1 + tv * w2c[1][c]
                        a2 = a2 + tv * w2c[2][c]
                    d0 = jnp.sum(a0)
                    d1 = jnp.sum(a1)
                    d2 = jnp.sum(a2)
                    orow = (be2r
                            + jnp.where(lane == 0, d0, 0.0)
                            + jnp.where(lane == 1, d1, 0.0)
                            + jnp.where(lane == 2, d2, 0.0))
                    outb[r, :] = orow

            pltpu.async_copy(outb, ep_out.at[pl.ds(off, _K)], osem)

        start(0, 0)

        @pl.loop(0, (t_steps + 1) // 2)
        def _(i):
            t0 = i * 2
            t1 = t0 + 1

            @pl.when(t1 < nb)
            def _():
                start(1, t1)

            @pl.when(t0 < nb)
            def _():
                finish(0, t0, drain=t0 >= 2)

            @pl.when(t0 + 2 < nb)
            def _():
                start(0, t0 + 2)

            @pl.when(t1 < nb)
            def _():
                finish(1, t1, drain=t1 >= 3)

        def odrain(slot):
            sdb, prow, qrow, outb, gsem, osem = slots[slot]
            pltpu.make_async_copy(outb, ep_out.at[pl.ds(0, _K)], osem).wait()

        odrain(0)

        @pl.when(nb >= 2)
        def _():
            odrain(1)

    kern = pl.kernel(
        body,
        out_type=jax.ShapeDtypeStruct((e, _LANES), jnp.float32),
        mesh=mesh,
        scratch_types=[
            pltpu.VMEM((2, _K), jnp.int32),
            pltpu.VMEM((2, _K), jnp.int32),
            pltpu.VMEM((_K, d), jnp.float32),
            pltpu.VMEM((_K, d), jnp.float32),
            pltpu.VMEM((_K, d), jnp.float32),
            pltpu.VMEM((_K, d), jnp.float32),
            pltpu.VMEM((_K, _LANES), jnp.float32),
            pltpu.VMEM((_K, _LANES), jnp.float32),
            pltpu.VMEM((3, d), jnp.float32),
            pltpu.VMEM((_LANES,), jnp.float32),
            pltpu.SemaphoreType.DMA,
            pltpu.SemaphoreType.DMA,
            pltpu.SemaphoreType.DMA,
            pltpu.SemaphoreType.DMA,
        ],
        compiler_params=_sc_compiler_params(),
    )
    return kern


def kernel(x, edge_index, W1, b1, gat_Wg, gat_att_src, gat_att_dst, gat_bias,
           Wp1, bp1, Wp2, bp2, We1, be1, We2, be2):
    n, d = x.shape
    e = edge_index.shape[1]
    num_layers = gat_Wg.shape[0]

    eidx32 = edge_index.astype(jnp.int32)
    eidx3 = eidx32.reshape(2, e // _K, _K).transpose(1, 0, 2)
    eidxg = eidx32.reshape(2, e // _KG, _KG).transpose(1, 0, 2)

    gat_kern, npad = _sc_gat_kernel(n, e, d)

    def sc_layer(l, xp, aT):
        a_s = aT[:, 0]
        a_d = aT[:, 1]
        gmax = jnp.max(a_s) + jnp.max(a_d)
        gmax = jnp.where(gmax >= 0.0, gmax, 0.2 * gmax)
        garr = jnp.full((_LANES,), gmax, jnp.float32)
        accs, dens = gat_kern(xp, a_s, a_d, garr, eidxg)
        acc0 = accs[:n]
        acc1 = accs[npad:npad + n]
        dnT = dens.reshape(_NC, npad)[:, :n].T
        return acc0, acc1, dnT

    def att2(l):
        return jnp.stack([gat_att_src[l], gat_att_dst[l]])

    h, xp, aT = _tc_input_pre(x, W1, b1.reshape(1, d), gat_Wg[0], att2(0))
    for l in range(num_layers - 1):
        acc0, acc1, dnT = sc_layer(l, xp, aT)
        h, xp, aT = _tc_epi_pre(h, acc0, acc1, dnT, gat_bias[l].reshape(1, d),
                                gat_Wg[l + 1], att2(l + 1))
    acc0, acc1, dnT = sc_layer(num_layers - 1, xp, aT)

    h3, P, Q = _tc_epi_pq(
        h, acc0, acc1, dnT, gat_bias[num_layers - 1].reshape(1, d),
        We1[:d], We1[d:], be1.reshape(1, d))

    edge_kern = _sc_edge_kernel(n, e, d)
    w2t = We2.T
    be2p = jnp.concatenate([be2, jnp.zeros((_LANES - 3,), jnp.float32)])
    ep16 = edge_kern(P, Q, w2t, be2p, eidx3)
    edge_pred = ep16[:, :3]

    node_pred, graph_emb = _tc_node_final(
        h3, Wp1, bp1.reshape(1, d), Wp2, bp2.reshape(1, d))

    return (node_pred, edge_pred, graph_emb)


# speedup vs baseline: 1.1080x; 1.1040x over previous
import dataclasses
import functools

import jax
import jax.numpy as jnp
from jax import lax
from jax.experimental import pallas as pl
from jax.experimental.pallas import tpu as pltpu
from jax.experimental.pallas import tpu_sc as plsc

_NC = 2
_NS = 16
_LANES = 16
_NW = _NC * _NS
_K = 128


def _cdiv(a, b):
    return (a + b - 1) // b


def _sc_compiler_params():
    cp = pltpu.CompilerParams()
    if "needs_layout_passes" in pltpu.CompilerParams.__dataclass_fields__:
        cp = dataclasses.replace(cp, needs_layout_passes=False)
    return cp


_ROW_BLK = 1000


def _row_spec(d):
    return pl.BlockSpec((_ROW_BLK, d), lambda i: (i, 0))


def _full_spec(r, c):
    return pl.BlockSpec((r, c), lambda i: (0, 0))


def _proj(xp, att):
    return lax.dot_general(
        xp, att, (((1,), (1,)), ((), ())),
        preferred_element_type=jnp.float32,
    )


def _agg(a0_ref, a1_ref, dn_ref):
    dn = (dn_ref[:, 0] + dn_ref[:, 1])[:, None]
    acc = a0_ref[...] + a1_ref[...]
    safe = jnp.where(dn > 0, dn, 1.0)
    return jnp.where(dn > 0, acc / safe, 0.0)


def _inpre_body(x_ref, w1_ref, b1_ref, wg_ref, att_ref, h_ref, xp_ref, a_ref):
    h = jnp.maximum(
        jnp.dot(x_ref[...], w1_ref[...], preferred_element_type=jnp.float32)
        + b1_ref[...], 0.0)
    h_ref[...] = h
    xp = jnp.dot(h, wg_ref[...], preferred_element_type=jnp.float32)
    xp_ref[...] = xp
    a_ref[...] = _proj(xp, att_ref[...])


def _tc_input_pre(x, W1, b1, Wg, att2):
    n, d = x.shape
    return pl.pallas_call(
        _inpre_body,
        grid=(n // _ROW_BLK,),
        in_specs=[_row_spec(d), _full_spec(d, d), _full_spec(1, d),
                  _full_spec(d, d), _full_spec(2, d)],
        out_specs=[_row_spec(d), _row_spec(d), _row_spec(2)],
        out_shape=[
            jax.ShapeDtypeStruct((n, d), jnp.float32),
            jax.ShapeDtypeStruct((n, d), jnp.float32),
            jax.ShapeDtypeStruct((n, 2), jnp.float32),
        ],
    )(x, W1, b1, Wg, att2)


def _epipre_body(h_ref, a0_ref, a1_ref, dn_ref, bg_ref, wg_ref, att_ref,
                 hn_ref, xp_ref, a_ref):
    h = jnp.maximum(h_ref[...] + _agg(a0_ref, a1_ref, dn_ref) + bg_ref[...],
                    0.0)
    hn_ref[...] = h
    xp = jnp.dot(h, wg_ref[...], preferred_element_type=jnp.float32)
    xp_ref[...] = xp
    a_ref[...] = _proj(xp, att_ref[...])


def _tc_epi_pre(h, acc0, acc1, dnT, bg, Wg, att2):
    n, d = h.shape
    return pl.pallas_call(
        _epipre_body,
        grid=(n // _ROW_BLK,),
        in_specs=[_row_spec(d), _row_spec(d), _row_spec(d), _row_spec(2),
                  _full_spec(1, d), _full_spec(d, d), _full_spec(2, d)],
        out_specs=[_row_spec(d), _row_spec(d), _row_spec(2)],
        out_shape=[
            jax.ShapeDtypeStruct((n, d), jnp.float32),
            jax.ShapeDtypeStruct((n, d), jnp.float32),
            jax.ShapeDtypeStruct((n, 2), jnp.float32),
        ],
    )(h, acc0, acc1, dnT, bg, Wg, att2)


def _epipq_body(h_ref, a0_ref, a1_ref, dn_ref, bg_ref, wea_ref, web_ref,
                be1_ref, h3_ref, p_ref, q_ref):
    h = jnp.maximum(h_ref[...] + _agg(a0_ref, a1_ref, dn_ref) + bg_ref[...],
                    0.0)
    h3_ref[...] = h
    p_ref[...] = (
        jnp.dot(h, wea_ref[...], preferred_element_type=jnp.float32)
        + be1_ref[...])
    q_ref[...] = jnp.dot(h, web_ref[...], preferred_element_type=jnp.float32)


def _tc_epi_pq(h, acc0, acc1, dnT, bg, We1a, We1b, be1):
    n, d = h.shape
    return pl.pallas_call(
        _epipq_body,
        grid=(n // _ROW_BLK,),
        in_specs=[_row_spec(d), _row_spec(d), _row_spec(d), _row_spec(2),
                  _full_spec(1, d), _full_spec(d, d), _full_spec(d, d),
                  _full_spec(1, d)],
        out_specs=[_row_spec(d), _row_spec(d), _row_spec(d)],
        out_shape=[
            jax.ShapeDtypeStruct((n, d), jnp.float32),
            jax.ShapeDtypeStruct((n, d), jnp.float32),
            jax.ShapeDtypeStruct((n, d), jnp.float32),
        ],
    )(h, acc0, acc1, dnT, bg, We1a, We1b, be1)


def _nodefin_body(h_ref, wp1_ref, bp1_ref, wp2_ref, bp2_ref, inv_n_ref,
                  np_ref, g_ref):
    h = h_ref[...]
    t = jnp.maximum(
        jnp.dot(h, wp1_ref[...], preferred_element_type=jnp.float32)
        + bp1_ref[...], 0.0)
    np_ref[...] = (
        jnp.dot(t, wp2_ref[...], preferred_element_type=jnp.float32)
        + bp2_ref[...])
    i = pl.program_id(0)

    @pl.when(i == 0)
    def _():
        g_ref[...] = jnp.zeros_like(g_ref)

    g_ref[...] += jnp.sum(h, axis=0, keepdims=True) * inv_n_ref[...]


def _tc_node_final(h3, Wp1, bp1, Wp2, bp2):
    n, d = h3.shape
    inv_n = jnp.full((1, 1), 1.0 / n, jnp.float32)
    return pl.pallas_call(
        _nodefin_body,
        grid=(n // _ROW_BLK,),
        in_specs=[_row_spec(d), _full_spec(d, d), _full_spec(1, d),
                  _full_spec(d, d), _full_spec(1, d), _full_spec(1, 1)],
        out_specs=[_row_spec(d), _full_spec(1, d)],
        out_shape=[
            jax.ShapeDtypeStruct((n, d), jnp.float32),
            jax.ShapeDtypeStruct((1, d), jnp.float32),
        ],
    )(h3, Wp1, bp1, Wp2, bp2, inv_n)


_KG = 64
_DEPTH = 4


@functools.lru_cache(maxsize=None)
def _sc_gat_kernel(n, e, d):
    rpt = _K * _cdiv(_cdiv(n, _NS), _K)
    npad = _NS * rpt
    nblk = e // _KG
    t_steps = _cdiv(nblk, _NW)
    nch = d // _LANES

    mesh = plsc.VectorSubcoreMesh(core_axis_name="c", subcore_axis_name="s")

    def body(xp_hbm, as_hbm, ad_hbm, g_hbm, eidx_hbm, acc_out, den_out,
             gvv, *rest):
        bufs = rest[:5 * _DEPTH]
        acc_sh, den_sh = rest[5 * _DEPTH:5 * _DEPTH + 2]
        sems = rest[5 * _DEPTH + 2:]
        slots = tuple(
            tuple(bufs[q + _DEPTH * k] for k in range(5))
            + (sems[q], sems[_DEPTH + q])
            for q in range(_DEPTH))

        cid = lax.axis_index("c")
        sid = lax.axis_index("s")
        wid = cid * _NS + sid

        pltpu.sync_copy(g_hbm, gvv)

        z16 = jnp.zeros((_LANES,), jnp.float32)
        rows0 = slots[0][4]
        exb0 = slots[0][3]

        @pl.loop(0, _KG)
        def _(r):
            for c in range(nch):
                rows0[r, pl.ds(c * _LANES, _LANES)] = z16

        for i in range(_KG // _LANES):
            exb0[pl.ds(i * _LANES, _LANES)] = z16

        zbase = sid * rpt
        zcp = []
        for k in range(rpt // _KG):
            zcp.append(pltpu.async_copy(
                rows0, acc_sh.at[pl.ds(zbase + k * _KG, _KG)], sems[0]))
            zcp.append(pltpu.async_copy(
                exb0, den_sh.at[pl.ds(zbase + k * _KG, _KG)], sems[1]))
        for c in zcp:
            c.wait()
        plsc.subcore_barrier()

        gvec = gvv[...]
        nb = (nblk - 1 - wid) // _NW + 1

        def issue(slot, t):
            sdb, avs, avd, exb, rows, gsem, ssem = slots[slot]
            @pl.when(t >= _DEPTH)
            def _():
                pltpu.make_async_copy(rows, acc_sh.at[sdb.at[1]], ssem).wait()
                pltpu.make_async_copy(exb, den_sh.at[sdb.at[1]], ssem).wait()
            blk = t * _NW + wid
            pltpu.sync_copy(eidx_hbm.at[blk], sdb)
            pltpu.async_copy(as_hbm.at[sdb.at[0]], avs, gsem)
            pltpu.async_copy(ad_hbm.at[sdb.at[1]], avd, gsem)
            pltpu.async_copy(xp_hbm.at[sdb.at[0]], rows, gsem)

        def compute(slot):
            sdb, avs, avd, exb, rows, gsem, ssem = slots[slot]
            pltpu.make_async_copy(as_hbm.at[sdb.at[0]], avs, gsem).wait()
            pltpu.make_async_copy(ad_hbm.at[sdb.at[1]], avd, gsem).wait()
            for j in range(_KG // _LANES):
                sl = pl.ds(j * _LANES, _LANES)
                ev = avs[sl] + avd[sl]
                ev = jnp.where(ev >= 0.0, ev, ev * 0.2)
                exb[sl] = jnp.exp(ev - gvec)
            pltpu.async_copy(exb, den_sh.at[sdb.at[1]], ssem, add=True)
            pltpu.make_async_copy(xp_hbm.at[sdb.at[0]], rows, gsem).wait()

            @pl.loop(0, _KG // _LANES)
            def _(jg):
                exv = exb[pl.ds(jg * _LANES, _LANES)]
                for rr in range(_LANES):
                    r = jg * _LANES + rr
                    exr = lax.gather(
                        exv,
                        jnp.full((_LANES, 1), rr, jnp.int32),
                        dimension_numbers=lax.GatherDimensionNumbers(
                            offset_dims=(),
                            collapsed_slice_dims=(0,),
                            start_index_map=(0,)),
                        slice_sizes=(1,),
                        mode=lax.GatherScatterMode.PROMISE_IN_BOUNDS)
                    for c in range(nch):
                        sl = pl.ds(c * _LANES, _LANES)
                        rows[r, sl] = rows[r, sl] * exr

            pltpu.async_copy(rows, acc_sh.at[sdb.at[1]], ssem, add=True)

        @pl.loop(0, (t_steps + 2 + _DEPTH - 1) // _DEPTH)
        def _(i):
            for q in range(_DEPTH):
                t = i * _DEPTH + q

                @pl.when(t < nb)
                def _(t=t, q=q):
                    issue(q, t)

                @pl.when((t >= 2) & (t - 2 < nb))
                def _(t=t, q=q):
                    compute((q + 2) % _DEPTH)

        for q in range(_DEPTH):
            sdb, avs, avd, exb, rows, gsem, ssem = slots[q]
            pltpu.make_async_copy(rows, acc_sh.at[sdb.at[1]], ssem).wait()
            pltpu.make_async_copy(exb, den_sh.at[sdb.at[1]], ssem).wait()

        plsc.subcore_barrier()
        ob = cid * npad + sid * rpt
        ocp = []
        for k in range(rpt // _K):
            ocp.append(pltpu.async_copy(
                acc_sh.at[pl.ds(zbase + k * _K, _K)],
                acc_out.at[pl.ds(ob + k * _K, _K)], sems[0]))
        ocp.append(pltpu.async_copy(
            den_sh.at[pl.ds(zbase, rpt)], den_out.at[pl.ds(ob, rpt)], sems[1]))
        for c in ocp:
            c.wait()

    kern = pl.kernel(
        body,
        out_type=[
            jax.ShapeDtypeStruct((_NC * npad, d), jnp.float32),
            jax.ShapeDtypeStruct((_NC * npad,), jnp.float32),
        ],
        mesh=mesh,
        scratch_types=(
            [pltpu.VMEM((_LANES,), jnp.float32)]
            + [pltpu.VMEM((2, _KG), jnp.int32)] * _DEPTH
            + [pltpu.VMEM((_KG,), jnp.float32)] * _DEPTH
            + [pltpu.VMEM((_KG,), jnp.float32)] * _DEPTH
            + [pltpu.VMEM((_KG,), jnp.float32)] * _DEPTH
            + [pltpu.VMEM((_KG, d), jnp.float32)] * _DEPTH
            + [pltpu.VMEM_SHARED((npad, d), jnp.float32),
               pltpu.VMEM_SHARED((npad,), jnp.float32)]
            + [pltpu.SemaphoreType.DMA] * (2 * _DEPTH)
        ),
        compiler_params=_sc_compiler_params(),
    )
    return kern, npad


@functools.lru_cache(maxsize=None)
def _sc_edge_kernel(n, e, d):
    nblk = e // _K
    t_steps = _cdiv(nblk, _NW)
    nch = d // _LANES

    mesh = plsc.VectorSubcoreMesh(core_axis_name="c", subcore_axis_name="s")

    def body(p_hbm, q_hbm, w2_hbm, be2_hbm, eidx_hbm, ep_out,
             sdb0, sdb1, prow0, prow1, qrow0, qrow1, outb0, outb1, w2v, be2v,
             gsem0, gsem1, osem0, osem1):
        cid = lax.axis_index("c")
        sid = lax.axis_index("s")
        wid = cid * _NS + sid

        pltpu.sync_copy(w2_hbm, w2v)
        pltpu.sync_copy(be2_hbm, be2v)
        be2r = be2v[...]
        lane = lax.iota(jnp.int32, _LANES)
        w2c = [[w2v[j, pl.ds(c * _LANES, _LANES)] for c in range(nch)]
               for j in range(3)]

        slots = ((sdb0, prow0, qrow0, outb0, gsem0, osem0),
                 (sdb1, prow1, qrow1, outb1, gsem1, osem1))
        nb = (nblk - 1 - wid) // _NW + 1

        def start(slot, t):
            sdb, prow, qrow, outb, gsem, osem = slots[slot]
            blk = t * _NW + wid
            pltpu.sync_copy(eidx_hbm.at[blk], sdb)
            pltpu.async_copy(p_hbm.at[sdb.at[0]], prow, gsem)
            pltpu.async_copy(q_hbm.at[sdb.at[1]], qrow, gsem)

        def finish(slot, t, drain):
            sdb, prow, qrow, outb, gsem, osem = slots[slot]
            pltpu.make_async_copy(p_hbm.at[sdb.at[0]], prow, gsem).wait()
            pltpu.make_async_copy(q_hbm.at[sdb.at[1]], qrow, gsem).wait()
            off = (t * _NW + wid) * _K
            if drain is True:
                pltpu.make_async_copy(outb, ep_out.at[pl.ds(off, _K)],
                                      osem).wait()
            elif drain is not False:
                @pl.when(drain)
                def _():
                    pltpu.make_async_copy(outb, ep_out.at[pl.ds(off, _K)],
                                          osem).wait()

            @pl.loop(0, _K // _LANES)
            def _(jg):
                for rr in range(_LANES):
                    r = jg * _LANES + rr
                    a0 = jnp.zeros((_LANES,), jnp.float32)
                    a1 = a0
                    a2 = a0
                    for c in range(nch):
                        sl = pl.ds(c * _LANES, _LANES)
                        tv = jnp.maximum(prow[r, sl] + qrow[r, sl], 0.0)
                        a0 = a0 + tv * w2c[0][c]
                        a1 = a1 + tv * w2c[1][c]
                        a2 = a2 + tv * w2c[2][c]
                    d0 = jnp.sum(a0)
                    d1 = jnp.sum(a1)
                    d2 = jnp.sum(a2)
                    orow = (be2r
                            + jnp.where(lane == 0, d0, 0.0)
                            + jnp.where(lane == 1, d1, 0.0)
                            + jnp.where(lane == 2, d2, 0.0))
                    outb[r, :] = orow

            pltpu.async_copy(outb, ep_out.at[pl.ds(off, _K)], osem)

        start(0, 0)

        @pl.loop(0, (t_steps + 1) // 2)
        def _(i):
            t0 = i * 2
            t1 = t0 + 1

            @pl.when(t1 < nb)
            def _():
                start(1, t1)

            @pl.when(t0 < nb)
            def _():
                finish(0, t0, drain=t0 >= 2)

            @pl.when(t0 + 2 < nb)
            def _():
                start(0, t0 + 2)

            @pl.when(t1 < nb)
            def _():
                finish(1, t1, drain=t1 >= 3)

        def odrain(slot):
            sdb, prow, qrow, outb, gsem, osem = slots[slot]
            pltpu.make_async_copy(outb, ep_out.at[pl.ds(0, _K)], osem).wait()

        odrain(0)

        @pl.when(nb >= 2)
        def _():
            odrain(1)

    kern = pl.kernel(
        body,
        out_type=jax.ShapeDtypeStruct((e, _LANES), jnp.float32),
        mesh=mesh,
        scratch_types=[
            pltpu.VMEM((2, _K), jnp.int32),
            pltpu.VMEM((2, _K), jnp.int32),
            pltpu.VMEM((_K, d), jnp.float32),
            pltpu.VMEM((_K, d), jnp.float32),
            pltpu.VMEM((_K, d), jnp.float32),
            pltpu.VMEM((_K, d), jnp.float32),
            pltpu.VMEM((_K, _LANES), jnp.float32),
            pltpu.VMEM((_K, _LANES), jnp.float32),
            pltpu.VMEM((3, d), jnp.float32),
            pltpu.VMEM((_LANES,), jnp.float32),
            pltpu.SemaphoreType.DMA,
            pltpu.SemaphoreType.DMA,
            pltpu.SemaphoreType.DMA,
            pltpu.SemaphoreType.DMA,
        ],
        compiler_params=_sc_compiler_params(),
    )
    return kern


def kernel(x, edge_index, W1, b1, gat_Wg, gat_att_src, gat_att_dst, gat_bias,
           Wp1, bp1, Wp2, bp2, We1, be1, We2, be2):
    n, d = x.shape
    e = edge_index.shape[1]
    num_layers = gat_Wg.shape[0]

    eidx32 = edge_index.astype(jnp.int32)
    eidx3 = eidx32.reshape(2, e // _K, _K).transpose(1, 0, 2)
    eidxg = eidx32.reshape(2, e // _KG, _KG).transpose(1, 0, 2)

    gat_kern, npad = _sc_gat_kernel(n, e, d)

    def sc_layer(l, xp, aT):
        a_s = aT[:, 0]
        a_d = aT[:, 1]
        gmax = jnp.max(a_s) + jnp.max(a_d)
        gmax = jnp.where(gmax >= 0.0, gmax, 0.2 * gmax)
        garr = jnp.full((_LANES,), gmax, jnp.float32)
        accs, dens = gat_kern(xp, a_s, a_d, garr, eidxg)
        acc0 = accs[:n]
        acc1 = accs[npad:npad + n]
        dnT = dens.reshape(_NC, npad)[:, :n].T
        return acc0, acc1, dnT

    def att2(l):
        return jnp.stack([gat_att_src[l], gat_att_dst[l]])

    h, xp, aT = _tc_input_pre(x, W1, b1.reshape(1, d), gat_Wg[0], att2(0))
    for l in range(num_layers - 1):
        acc0, acc1, dnT = sc_layer(l, xp, aT)
        h, xp, aT = _tc_epi_pre(h, acc0, acc1, dnT, gat_bias[l].reshape(1, d),
                                gat_Wg[l + 1], att2(l + 1))
    acc0, acc1, dnT = sc_layer(num_layers - 1, xp, aT)

    h3, P, Q = _tc_epi_pq(
        h, acc0, acc1, dnT, gat_bias[num_layers - 1].reshape(1, d),
        We1[:d], We1[d:], be1.reshape(1, d))

    edge_kern = _sc_edge_kernel(n, e, d)
    w2t = We2.T
    be2p = jnp.concatenate([be2, jnp.zeros((_LANES - 3,), jnp.float32)])
    ep16 = edge_kern(P, Q, w2t, be2p, eidx3)
    edge_pred = ep16[:, :3]

    node_pred, graph_emb = _tc_node_final(
        h3, Wp1, bp1.reshape(1, d), Wp2, bp2.reshape(1, d))

    return (node_pred, edge_pred, graph_emb)
